# baseline (speedup 1.0000x reference)
import functools

import jax
import jax.numpy as jnp
from jax import lax
from jax.experimental import pallas as pl
from jax.experimental.pallas import tpu as pltpu
from jax.experimental.pallas import tpu_sc as plsc

f32 = jnp.float32

N_NODES = 10000
N_EDGES = 320000
NC, NS = 2, 16
NW = NC * NS
EPW = N_EDGES // NW
CH = 80
STEPS = EPW // CH
K = 25
SLAB = K * CH
GROUPS = STEPS // K
NPT = N_NODES // NS

@functools.cache
def _sc_kernels():
    mesh = plsc.VectorSubcoreMesh(core_axis_name="c", subcore_axis_name="s",
                                  num_cores=NC, num_subcores=NS)
    cparams = pltpu.CompilerParams(use_tc_tiling_on_sc=False)

    @functools.partial(
        pl.kernel,
        out_type=jax.ShapeDtypeStruct((N_EDGES, 16), f32),
        mesh=mesh,
        scratch_types=[
            pltpu.VMEM((STEPS, CH), jnp.int32),
            pltpu.VMEM((SLAB, 16), f32),
            pltpu.SemaphoreType.DMA,
        ],
        compiler_params=cparams,
    )
    def sc_gather(table_hbm, idx_hbm, out_hbm, idx_v, slab, sem):
        cid = lax.axis_index("c")
        sid = lax.axis_index("s")
        wid = sid * NC + cid
        base = wid * EPW
        pltpu.sync_copy(idx_hbm.at[wid], idx_v)

        def group(g, carry):
            def fire(b, c2):
                pltpu.async_copy(table_hbm.at[idx_v.at[g * K + b]],
                                 slab.at[pl.ds(b * CH, CH)], sem)
                return c2

            lax.fori_loop(0, K, fire, 0)

            def drain(b, c2):
                pltpu.make_async_copy(table_hbm.at[idx_v.at[g * K + b]],
                                      slab.at[pl.ds(b * CH, CH)], sem).wait()
                return c2

            lax.fori_loop(0, K, drain, 0)
            pltpu.sync_copy(slab, out_hbm.at[pl.ds(base + g * SLAB, SLAB)])
            return carry

        lax.fori_loop(0, GROUPS, group, 0)

    @functools.partial(
        pl.kernel,
        out_type=jax.ShapeDtypeStruct((NC * N_NODES, 16), f32),
        mesh=mesh,
        scratch_types=[
            pltpu.VMEM_SHARED((N_NODES, 16), f32),
            pltpu.VMEM((STEPS, CH), jnp.int32),
            pltpu.VMEM((SLAB, 16), f32),
            pltpu.VMEM((NPT, 16), f32),
            pltpu.SemaphoreType.DMA,
        ],
        compiler_params=cparams,
    )
    def sc_scatter(msg_hbm, dst_hbm, zeros_hbm, out_hbm, accum, idx_v, slab,
                   rbuf, sem):
        cid = lax.axis_index("c")
        sid = lax.axis_index("s")
        wid = sid * NC + cid
        pltpu.sync_copy(zeros_hbm, rbuf)
        pltpu.sync_copy(rbuf, accum.at[pl.ds(sid * NPT, NPT)])
        plsc.subcore_barrier()
        pltpu.sync_copy(dst_hbm.at[wid], idx_v)

        def group(g, carry):
            pltpu.sync_copy(msg_hbm.at[pl.ds(wid * EPW + g * SLAB, SLAB)],
                            slab)

            def fire(b, c2):
                pltpu.async_copy(slab.at[pl.ds(b * CH, CH)],
                                 accum.at[idx_v.at[g * K + b]], sem, add=True)
                return c2

            lax.fori_loop(0, K, fire, 0)

            def drain(b, c2):
                pltpu.make_async_copy(slab.at[pl.ds(b * CH, CH)],
                                      accum.at[idx_v.at[g * K + b]],
                                      sem).wait()
                return c2

            lax.fori_loop(0, K, drain, 0)
            return carry

        lax.fori_loop(0, GROUPS, group, 0)
        plsc.subcore_barrier()
        pltpu.sync_copy(accum.at[pl.ds(sid * NPT, NPT)], rbuf)
        pltpu.sync_copy(rbuf,
                        out_hbm.at[pl.ds(cid * N_NODES + sid * NPT, NPT)])

    return sc_gather, sc_scatter


EP = N_EDGES // 8
BP = 400


bf16 = jnp.bfloat16


def _edge_body(ea_ref, xj_ref, w1_ref, b1_ref, w2_ref, b2_ref, r_ref, s_ref,
               out_ref):
    h = jnp.maximum(
        jnp.dot(ea_ref[...].astype(bf16), w1_ref[...],
                preferred_element_type=f32) + b1_ref[...], 0.0)
    w = jnp.maximum(
        jnp.dot(h.astype(bf16), w2_ref[...],
                preferred_element_type=f32) + b2_ref[...], 0.0)
    xje = jnp.dot(xj_ref[...].astype(bf16), r_ref[...],
                  preferred_element_type=f32)
    out_ref[...] = jnp.dot((w * xje).astype(bf16), s_ref[...],
                           preferred_element_type=f32)


def _tc_edge(ea_p, xj_p, w1bd, b1bd, w2bd, b2bd, rbd, sbd):
    return pl.pallas_call(
        _edge_body,
        grid=(EP // BP,),
        in_specs=[
            pl.BlockSpec((BP, 128), lambda i: (i, 0)),
            pl.BlockSpec((BP, 128), lambda i: (i, 0)),
            pl.BlockSpec((128, 512), lambda i: (0, 0)),
            pl.BlockSpec((1, 512), lambda i: (0, 0)),
            pl.BlockSpec((512, 2048), lambda i: (0, 0)),
            pl.BlockSpec((1, 2048), lambda i: (0, 0)),
            pl.BlockSpec((128, 2048), lambda i: (0, 0)),
            pl.BlockSpec((2048, 128), lambda i: (0, 0)),
        ],
        out_specs=pl.BlockSpec((BP, 128), lambda i: (i, 0)),
        out_shape=jax.ShapeDtypeStruct((EP, 128), f32),
    )(ea_p, xj_p, w1bd, b1bd, w2bd, b2bd, rbd, sbd)


def _make_combine(relu):
    def body(p0_ref, p1_ref, xin_ref, rt_ref, b_ref, out_ref):
        v = (p0_ref[...] + p1_ref[...]
             + jnp.dot(xin_ref[...], rt_ref[...], preferred_element_type=f32)
             + b_ref[...])
        out_ref[...] = jnp.maximum(v, 0.0) if relu else v
    return body


def _tc_combine(p, xin, rt, b, relu):
    return pl.pallas_call(
        _make_combine(relu),
        out_shape=jax.ShapeDtypeStruct((N_NODES, 16), f32),
    )(p[:N_NODES], p[N_NODES:], xin, rt, b)


def kernel(x, edge_index, edge_attr, n1W1, n1b1, n1W2, n1b2, root1, bias1,
           n2W1, n2b1, n2W2, n2b2, root2, bias2):
    ei = edge_index.astype(jnp.int32)
    src3 = ei[0].reshape(NW, STEPS, CH)
    dst3 = ei[1].reshape(NW, STEPS, CH)
    zeros = jnp.zeros((NPT, 16), f32)
    rmat = (jnp.arange(256)[None, :] // 16 == jnp.arange(16)[:, None]).astype(f32)
    smat = (jnp.arange(256)[:, None] % 16 == jnp.arange(16)[None, :]).astype(f32)
    eye8 = jnp.eye(8, dtype=f32)

    def kron8_bf(w):
        a, b = w.shape
        m = eye8[:, None, :, None] * w[None, :, None, :]
        return m.reshape(8 * a, 8 * b).astype(bf16)

    rbd = kron8_bf(rmat)
    sbd = kron8_bf(smat)
    w1bd_1 = kron8_bf(n1W1.T)
    w2bd_1 = kron8_bf(n1W2.T)
    b1bd_1 = jnp.tile(n1b1, 8).reshape(1, 512)
    b2bd_1 = jnp.tile(n1b2, 8).reshape(1, 2048)
    w1bd_2 = kron8_bf(n2W1.T)
    w2bd_2 = kron8_bf(n2W2.T)
    b1bd_2 = jnp.tile(n2b1, 8).reshape(1, 512)
    b2bd_2 = jnp.tile(n2b2, 8).reshape(1, 2048)
    ea_p = edge_attr.reshape(EP, 128)

    sc_gather, sc_scatter = _sc_kernels()

    xj1 = sc_gather(x, src3)
    msg1 = _tc_edge(ea_p, xj1.reshape(EP, 128), w1bd_1, b1bd_1,
                    w2bd_1, b2bd_1, rbd, sbd)
    p1 = sc_scatter(msg1.reshape(N_EDGES, 16), dst3, zeros)
    h = _tc_combine(p1, x, root1.T, bias1.reshape(1, 16), relu=True)

    xj2 = sc_gather(h, src3)
    msg2 = _tc_edge(ea_p, xj2.reshape(EP, 128), w1bd_2, b1bd_2,
                    w2bd_2, b2bd_2, rbd, sbd)
    p2 = sc_scatter(msg2.reshape(N_EDGES, 16), dst3, zeros)
    out = _tc_combine(p2, h, root2.T, bias2.reshape(1, 16), relu=False)
    return out


# speedup vs baseline: 1.1995x; 1.1995x over previous
import functools

import jax
import jax.numpy as jnp
from jax import lax
from jax.experimental import pallas as pl
from jax.experimental.pallas import tpu as pltpu
from jax.experimental.pallas import tpu_sc as plsc

f32 = jnp.float32

N_NODES = 10000
N_EDGES = 320000
NC, NS = 2, 16
NW = NC * NS
EPW = N_EDGES // NW
CH = 80
STEPS = EPW // CH
K = 25
SLAB = K * CH
GROUPS = STEPS // K
NPT = N_NODES // NS

@functools.cache
def _sc_kernels():
    mesh = plsc.VectorSubcoreMesh(core_axis_name="c", subcore_axis_name="s",
                                  num_cores=NC, num_subcores=NS)
    cparams = pltpu.CompilerParams(use_tc_tiling_on_sc=False)

    @functools.partial(
        pl.kernel,
        out_type=jax.ShapeDtypeStruct((N_EDGES, 16), f32),
        mesh=mesh,
        scratch_types=[
            pltpu.VMEM((STEPS, CH), jnp.int32),
            pltpu.VMEM((SLAB, 16), f32),
            pltpu.SemaphoreType.DMA,
        ],
        compiler_params=cparams,
    )
    def sc_gather(table_hbm, idx_hbm, out_hbm, idx_v, slab, sem):
        cid = lax.axis_index("c")
        sid = lax.axis_index("s")
        wid = sid * NC + cid
        base = wid * EPW
        pltpu.sync_copy(idx_hbm.at[wid], idx_v)

        def group(g, carry):
            def fire(b, c2):
                pltpu.async_copy(table_hbm.at[idx_v.at[g * K + b]],
                                 slab.at[pl.ds(b * CH, CH)], sem)
                return c2

            lax.fori_loop(0, K, fire, 0)

            def drain(b, c2):
                pltpu.make_async_copy(table_hbm.at[idx_v.at[g * K + b]],
                                      slab.at[pl.ds(b * CH, CH)], sem).wait()
                return c2

            lax.fori_loop(0, K, drain, 0)
            pltpu.sync_copy(slab, out_hbm.at[pl.ds(base + g * SLAB, SLAB)])
            return carry

        lax.fori_loop(0, GROUPS, group, 0)

    @functools.partial(
        pl.kernel,
        out_type=jax.ShapeDtypeStruct((NC * N_NODES, 16), f32),
        mesh=mesh,
        scratch_types=[
            pltpu.VMEM_SHARED((N_NODES, 16), f32),
            pltpu.VMEM((STEPS, CH), jnp.int32),
            pltpu.VMEM((SLAB, 16), f32),
            pltpu.VMEM((NPT, 16), f32),
            pltpu.SemaphoreType.DMA,
        ],
        compiler_params=cparams,
    )
    def sc_scatter(msg_hbm, dst_hbm, zeros_hbm, out_hbm, accum, idx_v, slab,
                   rbuf, sem):
        cid = lax.axis_index("c")
        sid = lax.axis_index("s")
        wid = sid * NC + cid
        pltpu.sync_copy(zeros_hbm, rbuf)
        pltpu.sync_copy(rbuf, accum.at[pl.ds(sid * NPT, NPT)])
        plsc.subcore_barrier()
        pltpu.sync_copy(dst_hbm.at[wid], idx_v)

        def group(g, carry):
            pltpu.sync_copy(msg_hbm.at[pl.ds(wid * EPW + g * SLAB, SLAB)],
                            slab)

            def fire(b, c2):
                pltpu.async_copy(slab.at[pl.ds(b * CH, CH)],
                                 accum.at[idx_v.at[g * K + b]], sem, add=True)
                return c2

            lax.fori_loop(0, K, fire, 0)

            def drain(b, c2):
                pltpu.make_async_copy(slab.at[pl.ds(b * CH, CH)],
                                      accum.at[idx_v.at[g * K + b]],
                                      sem).wait()
                return c2

            lax.fori_loop(0, K, drain, 0)
            return carry

        lax.fori_loop(0, GROUPS, group, 0)
        plsc.subcore_barrier()
        pltpu.sync_copy(accum.at[pl.ds(sid * NPT, NPT)], rbuf)
        pltpu.sync_copy(rbuf,
                        out_hbm.at[pl.ds(cid * N_NODES + sid * NPT, NPT)])

    return sc_gather, sc_scatter


EP = N_EDGES // 8
BP = 1000


bf16 = jnp.bfloat16


def _edge_body(ea_ref, xj_ref, w1_ref, b1_ref, w2_ref, b2_ref, r_ref, s_ref,
               out_ref):
    ea = ea_ref[...]
    xj = xj_ref[...]
    eas = [ea[:, 16 * j:16 * (j + 1)].astype(bf16) for j in range(8)]
    xjs = [xj[:, 16 * j:16 * (j + 1)].astype(bf16) for j in range(8)]
    hs = [jnp.maximum(
        jnp.dot(e, w1_ref[...], preferred_element_type=f32)
        + b1_ref[...], 0.0).astype(bf16) for e in eas]
    ws = [jnp.maximum(
        jnp.dot(hh, w2_ref[...], preferred_element_type=f32)
        + b2_ref[...], 0.0).astype(bf16) for hh in hs]
    xjes = [jnp.dot(xx, r_ref[...], preferred_element_type=f32).astype(bf16)
            for xx in xjs]
    for j in range(8):
        msg = jnp.dot(ws[j] * xjes[j], s_ref[...], preferred_element_type=f32)
        out_ref[:, 16 * j:16 * (j + 1)] = msg


def _tc_edge(ea_p, xj_p, w1t, b1, w2t, b2, rmat, smat):
    return pl.pallas_call(
        _edge_body,
        grid=(EP // BP,),
        in_specs=[
            pl.BlockSpec((BP, 128), lambda i: (i, 0)),
            pl.BlockSpec((BP, 128), lambda i: (i, 0)),
            pl.BlockSpec((16, 64), lambda i: (0, 0)),
            pl.BlockSpec((1, 64), lambda i: (0, 0)),
            pl.BlockSpec((64, 256), lambda i: (0, 0)),
            pl.BlockSpec((1, 256), lambda i: (0, 0)),
            pl.BlockSpec((16, 256), lambda i: (0, 0)),
            pl.BlockSpec((256, 16), lambda i: (0, 0)),
        ],
        out_specs=pl.BlockSpec((BP, 128), lambda i: (i, 0)),
        out_shape=jax.ShapeDtypeStruct((EP, 128), f32),
    )(ea_p, xj_p, w1t, b1, w2t, b2, rmat, smat)


def _make_combine(relu):
    def body(p0_ref, p1_ref, xin_ref, rt_ref, b_ref, out_ref):
        v = (p0_ref[...] + p1_ref[...]
             + jnp.dot(xin_ref[...], rt_ref[...], preferred_element_type=f32)
             + b_ref[...])
        out_ref[...] = jnp.maximum(v, 0.0) if relu else v
    return body


def _tc_combine(p, xin, rt, b, relu):
    return pl.pallas_call(
        _make_combine(relu),
        out_shape=jax.ShapeDtypeStruct((N_NODES, 16), f32),
    )(p[:N_NODES], p[N_NODES:], xin, rt, b)


def kernel(x, edge_index, edge_attr, n1W1, n1b1, n1W2, n1b2, root1, bias1,
           n2W1, n2b1, n2W2, n2b2, root2, bias2):
    ei = edge_index.astype(jnp.int32)
    src3 = ei[0].reshape(NW, STEPS, CH)
    dst3 = ei[1].reshape(NW, STEPS, CH)
    zeros = jnp.zeros((NPT, 16), f32)
    rmat = (jnp.arange(256)[None, :] // 16 == jnp.arange(16)[:, None]).astype(f32)
    smat = (jnp.arange(256)[:, None] % 16 == jnp.arange(16)[None, :]).astype(f32)
    rmat_bf = rmat.astype(bf16)
    smat_bf = smat.astype(bf16)
    w1_1 = n1W1.T.astype(bf16)
    w2_1 = n1W2.T.astype(bf16)
    b1_1 = n1b1.reshape(1, 64)
    b2_1 = n1b2.reshape(1, 256)
    w1_2 = n2W1.T.astype(bf16)
    w2_2 = n2W2.T.astype(bf16)
    b1_2 = n2b1.reshape(1, 64)
    b2_2 = n2b2.reshape(1, 256)
    ea_p = edge_attr.reshape(EP, 128)

    sc_gather, sc_scatter = _sc_kernels()

    xj1 = sc_gather(x, src3)
    msg1 = _tc_edge(ea_p, xj1.reshape(EP, 128), w1_1, b1_1,
                    w2_1, b2_1, rmat_bf, smat_bf)
    p1 = sc_scatter(msg1.reshape(N_EDGES, 16), dst3, zeros)
    h = _tc_combine(p1, x, root1.T, bias1.reshape(1, 16), relu=True)

    xj2 = sc_gather(h, src3)
    msg2 = _tc_edge(ea_p, xj2.reshape(EP, 128), w1_2, b1_2,
                    w2_2, b2_2, rmat_bf, smat_bf)
    p2 = sc_scatter(msg2.reshape(N_EDGES, 16), dst3, zeros)
    out = _tc_combine(p2, h, root2.T, bias2.reshape(1, 16), relu=False)
    return out


# speedup vs baseline: 1.2070x; 1.0063x over previous
import functools

import jax
import jax.numpy as jnp
from jax import lax
from jax.experimental import pallas as pl
from jax.experimental.pallas import tpu as pltpu
from jax.experimental.pallas import tpu_sc as plsc

f32 = jnp.float32

N_NODES = 10000
N_EDGES = 320000
NC, NS = 2, 16
NW = NC * NS
EPW = N_EDGES // NW
CH = 80
STEPS = EPW // CH
K = 25
SLAB = K * CH
GROUPS = STEPS // K
NPT = N_NODES // NS

@functools.cache
def _sc_kernels():
    mesh = plsc.VectorSubcoreMesh(core_axis_name="c", subcore_axis_name="s",
                                  num_cores=NC, num_subcores=NS)
    cparams = pltpu.CompilerParams(use_tc_tiling_on_sc=False)

    @functools.partial(
        pl.kernel,
        out_type=jax.ShapeDtypeStruct((N_EDGES // 8, 128), f32),
        mesh=mesh,
        scratch_types=[
            pltpu.VMEM((STEPS, CH), jnp.int32),
            pltpu.VMEM((SLAB, 16), f32),
            pltpu.SemaphoreType.DMA,
        ],
        compiler_params=cparams,
    )
    def sc_gather(table_hbm, idx_hbm, out_hbm, idx_v, slab, sem):
        cid = lax.axis_index("c")
        sid = lax.axis_index("s")
        wid = sid * NC + cid
        row0 = (wid % 4) * EPW
        col = wid // 4
        pltpu.sync_copy(idx_hbm.at[wid], idx_v)

        def group(g, carry):
            def fire(b, c2):
                pltpu.async_copy(table_hbm.at[idx_v.at[g * K + b]],
                                 slab.at[pl.ds(b * CH, CH)], sem)
                return c2

            lax.fori_loop(0, K, fire, 0)

            def drain(b, c2):
                pltpu.make_async_copy(table_hbm.at[idx_v.at[g * K + b]],
                                      slab.at[pl.ds(b * CH, CH)], sem).wait()
                return c2

            lax.fori_loop(0, K, drain, 0)
            pltpu.sync_copy(slab, out_hbm.at[pl.ds(row0 + g * SLAB, SLAB),
                                             pl.ds(col * 16, 16)])
            return carry

        lax.fori_loop(0, GROUPS, group, 0)

    @functools.partial(
        pl.kernel,
        out_type=jax.ShapeDtypeStruct((NC * N_NODES, 16), f32),
        mesh=mesh,
        scratch_types=[
            pltpu.VMEM_SHARED((N_NODES, 16), f32),
            pltpu.VMEM((STEPS, CH), jnp.int32),
            pltpu.VMEM((SLAB, 16), f32),
            pltpu.VMEM((NPT, 16), f32),
            pltpu.SemaphoreType.DMA,
        ],
        compiler_params=cparams,
    )
    def sc_scatter(msg_hbm, dst_hbm, zeros_hbm, out_hbm, accum, idx_v, slab,
                   rbuf, sem):
        cid = lax.axis_index("c")
        sid = lax.axis_index("s")
        wid = sid * NC + cid
        row0 = (wid % 4) * EPW
        col = wid // 4
        pltpu.sync_copy(zeros_hbm, rbuf)
        pltpu.sync_copy(rbuf, accum.at[pl.ds(sid * NPT, NPT)])
        plsc.subcore_barrier()
        pltpu.sync_copy(dst_hbm.at[wid], idx_v)

        def group(g, carry):
            pltpu.sync_copy(msg_hbm.at[pl.ds(row0 + g * SLAB, SLAB),
                                       pl.ds(col * 16, 16)], slab)

            def fire(b, c2):
                pltpu.async_copy(slab.at[pl.ds(b * CH, CH)],
                                 accum.at[idx_v.at[g * K + b]], sem, add=True)
                return c2

            lax.fori_loop(0, K, fire, 0)

            def drain(b, c2):
                pltpu.make_async_copy(slab.at[pl.ds(b * CH, CH)],
                                      accum.at[idx_v.at[g * K + b]],
                                      sem).wait()
                return c2

            lax.fori_loop(0, K, drain, 0)
            return carry

        lax.fori_loop(0, GROUPS, group, 0)
        plsc.subcore_barrier()
        pltpu.sync_copy(accum.at[pl.ds(sid * NPT, NPT)], rbuf)
        pltpu.sync_copy(rbuf,
                        out_hbm.at[pl.ds(cid * N_NODES + sid * NPT, NPT)])

    return sc_gather, sc_scatter


EP = N_EDGES // 8
BP = 1000


bf16 = jnp.bfloat16


def _edge_body(*refs):
    ea_refs = refs[:8]
    xj_ref, w1_ref, b1_ref, w2_ref, b2_ref, r_ref, s_ref, out_ref = refs[8:]
    xj = xj_ref[...]
    eas = [r[...].astype(bf16) for r in ea_refs]
    xjs = [xj[:, 16 * j:16 * (j + 1)].astype(bf16) for j in range(8)]
    hs = [jnp.maximum(
        jnp.dot(e, w1_ref[...], preferred_element_type=f32)
        + b1_ref[...], 0.0).astype(bf16) for e in eas]
    ws = [jnp.maximum(
        jnp.dot(hh, w2_ref[...], preferred_element_type=f32)
        + b2_ref[...], 0.0).astype(bf16) for hh in hs]
    xjes = [jnp.dot(xx, r_ref[...], preferred_element_type=f32).astype(bf16)
            for xx in xjs]
    for j in range(8):
        msg = jnp.dot(ws[j] * xjes[j], s_ref[...], preferred_element_type=f32)
        out_ref[:, 16 * j:16 * (j + 1)] = msg


def _make_ea_spec(j):
    return pl.BlockSpec((BP, 16), lambda i, j=j: (j * (EP // BP) + i, 0))


def _tc_edge(ea, xj_p, w1t, b1, w2t, b2, rmat, smat):
    return pl.pallas_call(
        _edge_body,
        grid=(EP // BP,),
        in_specs=[_make_ea_spec(j) for j in range(8)] + [
            pl.BlockSpec((BP, 128), lambda i: (i, 0)),
            pl.BlockSpec((16, 64), lambda i: (0, 0)),
            pl.BlockSpec((1, 64), lambda i: (0, 0)),
            pl.BlockSpec((64, 256), lambda i: (0, 0)),
            pl.BlockSpec((1, 256), lambda i: (0, 0)),
            pl.BlockSpec((16, 256), lambda i: (0, 0)),
            pl.BlockSpec((256, 16), lambda i: (0, 0)),
        ],
        out_specs=pl.BlockSpec((BP, 128), lambda i: (i, 0)),
        out_shape=jax.ShapeDtypeStruct((EP, 128), f32),
    )(*([ea] * 8), xj_p, w1t, b1, w2t, b2, rmat, smat)


def _make_combine(relu):
    def body(p0_ref, p1_ref, xin_ref, rt_ref, b_ref, out_ref):
        v = (p0_ref[...] + p1_ref[...]
             + jnp.dot(xin_ref[...], rt_ref[...], preferred_element_type=f32)
             + b_ref[...])
        out_ref[...] = jnp.maximum(v, 0.0) if relu else v
    return body


def _tc_combine(p, xin, rt, b, relu):
    return pl.pallas_call(
        _make_combine(relu),
        out_shape=jax.ShapeDtypeStruct((N_NODES, 16), f32),
    )(p[:N_NODES], p[N_NODES:], xin, rt, b)


def kernel(x, edge_index, edge_attr, n1W1, n1b1, n1W2, n1b2, root1, bias1,
           n2W1, n2b1, n2W2, n2b2, root2, bias2):
    ei = edge_index.astype(jnp.int32)
    src3 = ei[0].reshape(NW, STEPS, CH)
    dst3 = ei[1].reshape(NW, STEPS, CH)
    zeros = jnp.zeros((NPT, 16), f32)
    rmat = (jnp.arange(256)[None, :] // 16 == jnp.arange(16)[:, None]).astype(f32)
    smat = (jnp.arange(256)[:, None] % 16 == jnp.arange(16)[None, :]).astype(f32)
    rmat_bf = rmat.astype(bf16)
    smat_bf = smat.astype(bf16)
    w1_1 = n1W1.T.astype(bf16)
    w2_1 = n1W2.T.astype(bf16)
    b1_1 = n1b1.reshape(1, 64)
    b2_1 = n1b2.reshape(1, 256)
    w1_2 = n2W1.T.astype(bf16)
    w2_2 = n2W2.T.astype(bf16)
    b1_2 = n2b1.reshape(1, 64)
    b2_2 = n2b2.reshape(1, 256)
    sc_gather, sc_scatter = _sc_kernels()

    xj1 = sc_gather(x, src3)
    msg1 = _tc_edge(edge_attr, xj1, w1_1, b1_1, w2_1, b2_1, rmat_bf, smat_bf)
    p1 = sc_scatter(msg1, dst3, zeros)
    h = _tc_combine(p1, x, root1.T, bias1.reshape(1, 16), relu=True)

    xj2 = sc_gather(h, src3)
    msg2 = _tc_edge(edge_attr, xj2, w1_2, b1_2, w2_2, b2_2, rmat_bf, smat_bf)
    p2 = sc_scatter(msg2, dst3, zeros)
    out = _tc_combine(p2, h, root2.T, bias2.reshape(1, 16), relu=False)
    return out


# speedup vs baseline: 1.2279x; 1.0174x over previous
import functools

import jax
import jax.numpy as jnp
from jax import lax
from jax.experimental import pallas as pl
from jax.experimental.pallas import tpu as pltpu
from jax.experimental.pallas import tpu_sc as plsc

f32 = jnp.float32

N_NODES = 10000
N_EDGES = 320000
NC, NS = 2, 16
NW = NC * NS
EPW = N_EDGES // NW
CH = 80
STEPS = EPW // CH
K = 25
SLAB = K * CH
GROUPS = STEPS // K
NPT = N_NODES // NS

@functools.cache
def _sc_kernels():
    mesh = plsc.VectorSubcoreMesh(core_axis_name="c", subcore_axis_name="s",
                                  num_cores=NC, num_subcores=NS)
    cparams = pltpu.CompilerParams(use_tc_tiling_on_sc=False)

    @functools.partial(
        pl.kernel,
        out_type=jax.ShapeDtypeStruct((N_EDGES // 8, 128), f32),
        mesh=mesh,
        scratch_types=[
            pltpu.VMEM((STEPS, CH), jnp.int32),
            pltpu.VMEM((SLAB, 16), f32),
            pltpu.SemaphoreType.DMA,
        ],
        compiler_params=cparams,
    )
    def sc_gather(table_hbm, idx_hbm, out_hbm, idx_v, slab, sem):
        cid = lax.axis_index("c")
        sid = lax.axis_index("s")
        wid = sid * NC + cid
        row0 = (wid % 4) * EPW
        col = wid // 4
        pltpu.sync_copy(idx_hbm.at[wid], idx_v)

        def group(g, carry):
            def fire(b, c2):
                pltpu.async_copy(table_hbm.at[idx_v.at[g * K + b]],
                                 slab.at[pl.ds(b * CH, CH)], sem)
                return c2

            lax.fori_loop(0, K, fire, 0)

            def drain(b, c2):
                pltpu.make_async_copy(table_hbm.at[idx_v.at[g * K + b]],
                                      slab.at[pl.ds(b * CH, CH)], sem).wait()
                return c2

            lax.fori_loop(0, K, drain, 0)
            pltpu.sync_copy(slab, out_hbm.at[pl.ds(row0 + g * SLAB, SLAB),
                                             pl.ds(col * 16, 16)])
            return carry

        lax.fori_loop(0, GROUPS, group, 0)

    @functools.partial(
        pl.kernel,
        out_type=jax.ShapeDtypeStruct((NC * N_NODES, 16), f32),
        mesh=mesh,
        scratch_types=[
            pltpu.VMEM_SHARED((N_NODES, 16), f32),
            pltpu.VMEM((STEPS, CH), jnp.int32),
            pltpu.VMEM((SLAB, 16), f32),
            pltpu.VMEM((NPT, 16), f32),
            pltpu.SemaphoreType.DMA,
        ],
        compiler_params=cparams,
    )
    def sc_scatter(msg_hbm, dst_hbm, zeros_hbm, out_hbm, accum, idx_v, slab,
                   rbuf, sem):
        cid = lax.axis_index("c")
        sid = lax.axis_index("s")
        wid = sid * NC + cid
        row0 = (wid % 4) * EPW
        col = wid // 4
        pltpu.sync_copy(zeros_hbm, rbuf)
        pltpu.sync_copy(rbuf, accum.at[pl.ds(sid * NPT, NPT)])
        plsc.subcore_barrier()
        pltpu.sync_copy(dst_hbm.at[wid], idx_v)

        def group(g, carry):
            pltpu.sync_copy(msg_hbm.at[pl.ds(row0 + g * SLAB, SLAB),
                                       pl.ds(col * 16, 16)], slab)

            def fire(b, c2):
                pltpu.async_copy(slab.at[pl.ds(b * CH, CH)],
                                 accum.at[idx_v.at[g * K + b]], sem, add=True)
                return c2

            lax.fori_loop(0, K, fire, 0)

            def drain(b, c2):
                pltpu.make_async_copy(slab.at[pl.ds(b * CH, CH)],
                                      accum.at[idx_v.at[g * K + b]],
                                      sem).wait()
                return c2

            lax.fori_loop(0, K, drain, 0)
            return carry

        lax.fori_loop(0, GROUPS, group, 0)
        plsc.subcore_barrier()
        pltpu.sync_copy(accum.at[pl.ds(sid * NPT, NPT)], rbuf)
        pltpu.sync_copy(rbuf,
                        out_hbm.at[pl.ds(cid * N_NODES + sid * NPT, NPT)])

    return sc_gather, sc_scatter


EP = N_EDGES // 8
BP = 1000


bf16 = jnp.bfloat16


def _edge_body(*refs):
    ea_refs = refs[:8]
    xj_ref, w1_ref, b1_ref, w2_ref, b2_ref, r_ref, s_ref, out_ref = refs[8:]
    xj = xj_ref[...]
    eas = [r[...].astype(bf16) for r in ea_refs]
    xjs = [xj[:, 16 * j:16 * (j + 1)].astype(bf16) for j in range(8)]
    hs = [jnp.maximum(
        jnp.dot(e, w1_ref[...], preferred_element_type=f32)
        + b1_ref[...], 0.0).astype(bf16) for e in eas]
    ws = [jnp.maximum(
        jnp.dot(hh, w2_ref[...], preferred_element_type=f32)
        + b2_ref[...], 0.0).astype(bf16) for hh in hs]
    xjes = [jnp.dot(xx, r_ref[...], preferred_element_type=f32).astype(bf16)
            for xx in xjs]
    for j in range(8):
        msg = jnp.dot(ws[j] * xjes[j], s_ref[...], preferred_element_type=f32)
        out_ref[:, 16 * j:16 * (j + 1)] = msg


def _make_ea_spec(j):
    return pl.BlockSpec((BP, 16), lambda i, j=j: (j * (EP // BP) + i, 0))


def _tc_edge(ea, xj_p, w1t, b1, w2t, b2, rmat, smat):
    return pl.pallas_call(
        _edge_body,
        grid=(EP // BP,),
        in_specs=[_make_ea_spec(j) for j in range(8)] + [
            pl.BlockSpec((BP, 128), lambda i: (i, 0)),
            pl.BlockSpec((16, 64), lambda i: (0, 0)),
            pl.BlockSpec((1, 64), lambda i: (0, 0)),
            pl.BlockSpec((64, 256), lambda i: (0, 0)),
            pl.BlockSpec((1, 256), lambda i: (0, 0)),
            pl.BlockSpec((16, 256), lambda i: (0, 0)),
            pl.BlockSpec((256, 16), lambda i: (0, 0)),
        ],
        out_specs=pl.BlockSpec((BP, 128), lambda i: (i, 0)),
        out_shape=jax.ShapeDtypeStruct((EP, 128), f32),
    )(*([ea] * 8), xj_p, w1t, b1, w2t, b2, rmat, smat)


NPK = N_NODES // 8


def _make_combine(relu):
    def body(p0_ref, p1_ref, xin_ref, rt_ref, b_ref, out_ref):
        v = (p0_ref[...] + p1_ref[...]
             + jnp.dot(xin_ref[...].astype(bf16), rt_ref[...],
                       preferred_element_type=f32)
             + b_ref[...])
        out_ref[...] = jnp.maximum(v, 0.0) if relu else v
    return body


def _tc_combine(p, xin_p, rt_bd, b_bd, relu):
    pp = p.reshape(2 * NPK, 128)
    return pl.pallas_call(
        _make_combine(relu),
        out_shape=jax.ShapeDtypeStruct((NPK, 128), f32),
    )(pp[:NPK], pp[NPK:], xin_p, rt_bd, b_bd)


def kernel(x, edge_index, edge_attr, n1W1, n1b1, n1W2, n1b2, root1, bias1,
           n2W1, n2b1, n2W2, n2b2, root2, bias2):
    ei = edge_index.astype(jnp.int32)
    src3 = ei[0].reshape(NW, STEPS, CH)
    dst3 = ei[1].reshape(NW, STEPS, CH)
    zeros = jnp.zeros((NPT, 16), f32)
    rmat = (jnp.arange(256)[None, :] // 16 == jnp.arange(16)[:, None]).astype(f32)
    smat = (jnp.arange(256)[:, None] % 16 == jnp.arange(16)[None, :]).astype(f32)
    rmat_bf = rmat.astype(bf16)
    smat_bf = smat.astype(bf16)
    w1_1 = n1W1.T.astype(bf16)
    w2_1 = n1W2.T.astype(bf16)
    b1_1 = n1b1.reshape(1, 64)
    b2_1 = n1b2.reshape(1, 256)
    w1_2 = n2W1.T.astype(bf16)
    w2_2 = n2W2.T.astype(bf16)
    b1_2 = n2b1.reshape(1, 64)
    b2_2 = n2b2.reshape(1, 256)
    eye8 = jnp.eye(8, dtype=f32)

    def kron8_bf(w):
        a, b = w.shape
        m = eye8[:, None, :, None] * w[None, :, None, :]
        return m.reshape(8 * a, 8 * b).astype(bf16)

    rt1_bd = kron8_bf(root1.T)
    rt2_bd = kron8_bf(root2.T)
    bias1_bd = jnp.tile(bias1, 8).reshape(1, 128)
    bias2_bd = jnp.tile(bias2, 8).reshape(1, 128)
    x_p = x.reshape(NPK, 128)

    sc_gather, sc_scatter = _sc_kernels()

    xj1 = sc_gather(x, src3)
    msg1 = _tc_edge(edge_attr, xj1, w1_1, b1_1, w2_1, b2_1, rmat_bf, smat_bf)
    p1 = sc_scatter(msg1, dst3, zeros)
    h_p = _tc_combine(p1, x_p, rt1_bd, bias1_bd, relu=True)
    h = h_p.reshape(N_NODES, 16)

    xj2 = sc_gather(h, src3)
    msg2 = _tc_edge(edge_attr, xj2, w1_2, b1_2, w2_2, b2_2, rmat_bf, smat_bf)
    p2 = sc_scatter(msg2, dst3, zeros)
    out_p = _tc_combine(p2, h_p, rt2_bd, bias2_bd, relu=False)
    return out_p.reshape(N_NODES, 16)


# speedup vs baseline: 1.2521x; 1.0197x over previous
import functools

import jax
import jax.numpy as jnp
from jax import lax
from jax.experimental import pallas as pl
from jax.experimental.pallas import tpu as pltpu
from jax.experimental.pallas import tpu_sc as plsc

f32 = jnp.float32

N_NODES = 10000
N_EDGES = 320000
NC, NS = 2, 16
NW = NC * NS
EPW = N_EDGES // NW
CH = 80
STEPS = EPW // CH
K = 25
SLAB = K * CH
GROUPS = STEPS // K
NPT = N_NODES // NS

@functools.cache
def _sc_kernels():
    mesh = plsc.VectorSubcoreMesh(core_axis_name="c", subcore_axis_name="s",
                                  num_cores=NC, num_subcores=NS)
    cparams = pltpu.CompilerParams(use_tc_tiling_on_sc=False)

    @functools.partial(
        pl.kernel,
        out_type=jax.ShapeDtypeStruct((N_EDGES // 8, 128), f32),
        mesh=mesh,
        scratch_types=[
            pltpu.VMEM((STEPS, CH), jnp.int32),
            pltpu.VMEM((SLAB, 16), f32),
            pltpu.SemaphoreType.DMA,
        ],
        compiler_params=cparams,
    )
    def sc_gather(table_hbm, idx_hbm, out_hbm, idx_v, slab, sem):
        cid = lax.axis_index("c")
        sid = lax.axis_index("s")
        wid = sid * NC + cid
        row0 = (wid % 4) * EPW
        col = wid // 4
        pltpu.sync_copy(idx_hbm.at[wid], idx_v)

        def group(g, carry):
            def fire(b, c2):
                pltpu.async_copy(table_hbm.at[idx_v.at[g * K + b]],
                                 slab.at[pl.ds(b * CH, CH)], sem)
                return c2

            lax.fori_loop(0, K, fire, 0)

            def drain(b, c2):
                pltpu.make_async_copy(table_hbm.at[idx_v.at[g * K + b]],
                                      slab.at[pl.ds(b * CH, CH)], sem).wait()
                return c2

            lax.fori_loop(0, K, drain, 0)
            pltpu.sync_copy(slab, out_hbm.at[pl.ds(row0 + g * SLAB, SLAB),
                                             pl.ds(col * 16, 16)])
            return carry

        lax.fori_loop(0, GROUPS, group, 0)

    @functools.partial(
        pl.kernel,
        out_type=jax.ShapeDtypeStruct((NC * N_NODES, 16), f32),
        mesh=mesh,
        scratch_types=[
            pltpu.VMEM_SHARED((N_NODES, 16), f32),
            pltpu.VMEM((STEPS, CH), jnp.int32),
            pltpu.VMEM((SLAB, 16), f32),
            pltpu.VMEM((NPT, 16), f32),
            pltpu.SemaphoreType.DMA,
        ],
        compiler_params=cparams,
    )
    def sc_scatter(msg_hbm, dst_hbm, zeros_hbm, out_hbm, accum, idx_v, slab,
                   rbuf, sem):
        cid = lax.axis_index("c")
        sid = lax.axis_index("s")
        wid = sid * NC + cid
        row0 = (wid % 4) * EPW
        col = wid // 4
        pltpu.sync_copy(zeros_hbm, rbuf)
        pltpu.sync_copy(rbuf, accum.at[pl.ds(sid * NPT, NPT)])
        plsc.subcore_barrier()
        pltpu.sync_copy(dst_hbm.at[wid], idx_v)

        def group(g, carry):
            pltpu.sync_copy(msg_hbm.at[pl.ds(row0 + g * SLAB, SLAB),
                                       pl.ds(col * 16, 16)], slab)

            def fire(b, c2):
                pltpu.async_copy(slab.at[pl.ds(b * CH, CH)],
                                 accum.at[idx_v.at[g * K + b]], sem, add=True)
                return c2

            lax.fori_loop(0, K, fire, 0)

            def drain(b, c2):
                pltpu.make_async_copy(slab.at[pl.ds(b * CH, CH)],
                                      accum.at[idx_v.at[g * K + b]],
                                      sem).wait()
                return c2

            lax.fori_loop(0, K, drain, 0)
            return carry

        lax.fori_loop(0, GROUPS, group, 0)
        plsc.subcore_barrier()
        pltpu.sync_copy(accum.at[pl.ds(sid * NPT, NPT)], rbuf)
        pltpu.sync_copy(rbuf,
                        out_hbm.at[pl.ds(cid * N_NODES + sid * NPT, NPT)])

    return sc_gather, sc_scatter


EP = N_EDGES // 8
BP = 2000


bf16 = jnp.bfloat16


def _edge_body(*refs):
    ea_refs = refs[:8]
    xj_ref, w1_ref, b1_ref, w2_ref, b2_ref, r_ref, s_ref, out_ref = refs[8:]
    xj = xj_ref[...]
    eas = [r[...].astype(bf16) for r in ea_refs]
    xjs = [xj[:, 16 * j:16 * (j + 1)].astype(bf16) for j in range(8)]
    hs = [jnp.maximum(
        jnp.dot(e, w1_ref[...], preferred_element_type=f32)
        + b1_ref[...], 0.0).astype(bf16) for e in eas]
    ws = [jnp.maximum(
        jnp.dot(hh, w2_ref[...], preferred_element_type=f32)
        + b2_ref[...], 0.0).astype(bf16) for hh in hs]
    xjes = [jnp.dot(xx, r_ref[...], preferred_element_type=f32).astype(bf16)
            for xx in xjs]
    for j in range(8):
        msg = jnp.dot(ws[j] * xjes[j], s_ref[...], preferred_element_type=f32)
        out_ref[:, 16 * j:16 * (j + 1)] = msg


def _make_ea_spec(j):
    return pl.BlockSpec((BP, 16), lambda i, j=j: (j * (EP // BP) + i, 0))


def _tc_edge(ea, xj_p, w1t, b1, w2t, b2, rmat, smat):
    return pl.pallas_call(
        _edge_body,
        grid=(EP // BP,),
        in_specs=[_make_ea_spec(j) for j in range(8)] + [
            pl.BlockSpec((BP, 128), lambda i: (i, 0)),
            pl.BlockSpec((16, 64), lambda i: (0, 0)),
            pl.BlockSpec((1, 64), lambda i: (0, 0)),
            pl.BlockSpec((64, 256), lambda i: (0, 0)),
            pl.BlockSpec((1, 256), lambda i: (0, 0)),
            pl.BlockSpec((16, 256), lambda i: (0, 0)),
            pl.BlockSpec((256, 16), lambda i: (0, 0)),
        ],
        out_specs=pl.BlockSpec((BP, 128), lambda i: (i, 0)),
        out_shape=jax.ShapeDtypeStruct((EP, 128), f32),
    )(*([ea] * 8), xj_p, w1t, b1, w2t, b2, rmat, smat)


NPK = N_NODES // 8


def _make_combine(relu):
    def body(p0_ref, p1_ref, xin_ref, rt_ref, b_ref, out_ref):
        v = (p0_ref[...] + p1_ref[...]
             + jnp.dot(xin_ref[...].astype(bf16), rt_ref[...],
                       preferred_element_type=f32)
             + b_ref[...])
        out_ref[...] = jnp.maximum(v, 0.0) if relu else v
    return body


def _tc_combine(p, xin_p, rt_bd, b_bd, relu):
    pp = p.reshape(2 * NPK, 128)
    return pl.pallas_call(
        _make_combine(relu),
        out_shape=jax.ShapeDtypeStruct((NPK, 128), f32),
    )(pp[:NPK], pp[NPK:], xin_p, rt_bd, b_bd)


def kernel(x, edge_index, edge_attr, n1W1, n1b1, n1W2, n1b2, root1, bias1,
           n2W1, n2b1, n2W2, n2b2, root2, bias2):
    ei = edge_index.astype(jnp.int32)
    src3 = ei[0].reshape(NW, STEPS, CH)
    dst3 = ei[1].reshape(NW, STEPS, CH)
    zeros = jnp.zeros((NPT, 16), f32)
    rmat = (jnp.arange(256)[None, :] // 16 == jnp.arange(16)[:, None]).astype(f32)
    smat = (jnp.arange(256)[:, None] % 16 == jnp.arange(16)[None, :]).astype(f32)
    rmat_bf = rmat.astype(bf16)
    smat_bf = smat.astype(bf16)
    w1_1 = n1W1.T.astype(bf16)
    w2_1 = n1W2.T.astype(bf16)
    b1_1 = n1b1.reshape(1, 64)
    b2_1 = n1b2.reshape(1, 256)
    w1_2 = n2W1.T.astype(bf16)
    w2_2 = n2W2.T.astype(bf16)
    b1_2 = n2b1.reshape(1, 64)
    b2_2 = n2b2.reshape(1, 256)
    eye8 = jnp.eye(8, dtype=f32)

    def kron8_bf(w):
        a, b = w.shape
        m = eye8[:, None, :, None] * w[None, :, None, :]
        return m.reshape(8 * a, 8 * b).astype(bf16)

    rt1_bd = kron8_bf(root1.T)
    rt2_bd = kron8_bf(root2.T)
    bias1_bd = jnp.tile(bias1, 8).reshape(1, 128)
    bias2_bd = jnp.tile(bias2, 8).reshape(1, 128)
    x_p = x.reshape(NPK, 128)

    sc_gather, sc_scatter = _sc_kernels()

    xj1 = sc_gather(x, src3)
    msg1 = _tc_edge(edge_attr, xj1, w1_1, b1_1, w2_1, b2_1, rmat_bf, smat_bf)
    p1 = sc_scatter(msg1, dst3, zeros)
    h_p = _tc_combine(p1, x_p, rt1_bd, bias1_bd, relu=True)
    h = h_p.reshape(N_NODES, 16)

    xj2 = sc_gather(h, src3)
    msg2 = _tc_edge(edge_attr, xj2, w1_2, b1_2, w2_2, b2_2, rmat_bf, smat_bf)
    p2 = sc_scatter(msg2, dst3, zeros)
    out_p = _tc_combine(p2, h_p, rt2_bd, bias2_bd, relu=False)
    return out_p.reshape(N_NODES, 16)
